# baseline (speedup 1.0000x reference)
import functools

import jax
import jax.numpy as jnp
from jax.experimental import pallas as pl
from jax.experimental.pallas import tpu as pltpu

N_EXPERTS = 8
TOP_K = 2
D_MODEL = 1024
D_FF = 2048
BF = 2048
NF = D_FF // BF


def _moe_kernel(x_ref, wr_ref, wg_ref, wu_ref, wd_ref,
                out_ref, aux_ref, comb_ref):
    e = pl.program_id(0)
    f = pl.program_id(1)
    xf = x_ref[...]

    @pl.when((e == 0) & (f == 0))
    def _routing():
        logits = jax.lax.dot_general(
            xf, wr_ref[...], (((1,), (1,)), ((), ())),
            preferred_element_type=jnp.float32)
        m = jnp.max(logits, axis=-1, keepdims=True)
        ex = jnp.exp(logits - m)
        scores = ex / jnp.sum(ex, axis=-1, keepdims=True)
        iota = jax.lax.broadcasted_iota(jnp.int32, scores.shape, 1)
        m1 = jnp.max(scores, axis=-1, keepdims=True)
        i1 = jnp.min(jnp.where(scores == m1, iota, N_EXPERTS),
                     axis=-1, keepdims=True)
        sel1 = iota == i1
        masked = jnp.where(sel1, -jnp.inf, scores)
        m2 = jnp.max(masked, axis=-1, keepdims=True)
        i2 = jnp.min(jnp.where(masked == m2, iota, N_EXPERTS),
                     axis=-1, keepdims=True)
        sel = sel1 | (iota == i2)
        comb_ref[...] = jnp.where(sel, scores, 0.0) / (m1 + m2)
        probs = jnp.mean(scores, axis=0, keepdims=True)
        fracs = jnp.mean(sel.astype(jnp.float32), axis=0, keepdims=True)
        aux_ref[...] = N_EXPERTS * jnp.sum(probs * fracs, keepdims=True)
        out_ref[...] = jnp.zeros_like(out_ref)

    x16 = xf.astype(jnp.bfloat16)
    g = jax.lax.dot_general(x16, wg_ref[0].astype(jnp.bfloat16),
                            (((1,), (1,)), ((), ())),
                            preferred_element_type=jnp.float32)
    u = jax.lax.dot_general(x16, wu_ref[0].astype(jnp.bfloat16),
                            (((1,), (1,)), ((), ())),
                            preferred_element_type=jnp.float32)
    h = (g * jax.lax.logistic(g)) * u
    iota = jax.lax.broadcasted_iota(jnp.int32, (comb_ref.shape[0], N_EXPERTS), 1)
    c_e = jnp.sum(jnp.where(iota == e, comb_ref[...], 0.0),
                  axis=-1, keepdims=True)
    hs = (h * c_e).astype(jnp.bfloat16)
    y = jax.lax.dot_general(hs, wd_ref[0].astype(jnp.bfloat16),
                            (((1,), (1,)), ((), ())),
                            preferred_element_type=jnp.float32)
    out_ref[...] += y


@functools.partial(jax.jit, static_argnums=())
def kernel(x, W_router, W_gate, W_up, W_down):
    Bx, Tx, D = x.shape
    xf = x.reshape(-1, D)
    n = xf.shape[0]
    out, aux = pl.pallas_call(
        _moe_kernel,
        grid=(N_EXPERTS, NF),
        in_specs=[
            pl.BlockSpec((n, D), lambda e, f: (0, 0)),
            pl.BlockSpec((N_EXPERTS, D), lambda e, f: (0, 0)),
            pl.BlockSpec((1, BF, D), lambda e, f: (e, f, 0)),
            pl.BlockSpec((1, BF, D), lambda e, f: (e, f, 0)),
            pl.BlockSpec((1, D, BF), lambda e, f: (e, 0, f)),
        ],
        out_specs=[
            pl.BlockSpec((n, D), lambda e, f: (0, 0)),
            pl.BlockSpec((1, 1), lambda e, f: (0, 0)),
        ],
        out_shape=[
            jax.ShapeDtypeStruct((n, D), jnp.float32),
            jax.ShapeDtypeStruct((1, 1), jnp.float32),
        ],
        scratch_shapes=[pltpu.VMEM((n, N_EXPERTS), jnp.float32)],
        compiler_params=pltpu.CompilerParams(
            dimension_semantics=("arbitrary", "arbitrary"),
        ),
    )(xf, W_router, W_gate, W_up, W_down)
    return out.reshape(Bx, Tx, D), aux[0, 0]


# speedup vs baseline: 1.0689x; 1.0689x over previous
import functools

import jax
import jax.numpy as jnp
from jax.experimental import pallas as pl
from jax.experimental.pallas import tpu as pltpu

N_EXPERTS = 8
TOP_K = 2
D_MODEL = 1024
D_FF = 2048
BF = 1024
NF = D_FF // BF


def _moe_kernel(x_ref, wr_ref, wg_ref, wu_ref, wd_ref,
                out_ref, aux_ref, comb_ref):
    e = pl.program_id(0)
    f = pl.program_id(1)
    xf = x_ref[...]

    @pl.when((e == 0) & (f == 0))
    def _routing():
        logits = jax.lax.dot_general(
            xf, wr_ref[...], (((1,), (1,)), ((), ())),
            preferred_element_type=jnp.float32)
        m = jnp.max(logits, axis=-1, keepdims=True)
        ex = jnp.exp(logits - m)
        scores = ex / jnp.sum(ex, axis=-1, keepdims=True)
        iota = jax.lax.broadcasted_iota(jnp.int32, scores.shape, 1)
        m1 = jnp.max(scores, axis=-1, keepdims=True)
        i1 = jnp.min(jnp.where(scores == m1, iota, N_EXPERTS),
                     axis=-1, keepdims=True)
        sel1 = iota == i1
        masked = jnp.where(sel1, -jnp.inf, scores)
        m2 = jnp.max(masked, axis=-1, keepdims=True)
        i2 = jnp.min(jnp.where(masked == m2, iota, N_EXPERTS),
                     axis=-1, keepdims=True)
        sel = sel1 | (iota == i2)
        comb_ref[...] = jnp.where(sel, scores, 0.0) / (m1 + m2)
        probs = jnp.mean(scores, axis=0, keepdims=True)
        fracs = jnp.mean(sel.astype(jnp.float32), axis=0, keepdims=True)
        aux_ref[...] = N_EXPERTS * jnp.sum(probs * fracs, keepdims=True)
        out_ref[...] = jnp.zeros_like(out_ref)

    x16 = xf.astype(jnp.bfloat16)
    g = jax.lax.dot_general(x16, wg_ref[0].astype(jnp.bfloat16),
                            (((1,), (1,)), ((), ())),
                            preferred_element_type=jnp.float32)
    u = jax.lax.dot_general(x16, wu_ref[0].astype(jnp.bfloat16),
                            (((1,), (1,)), ((), ())),
                            preferred_element_type=jnp.float32)
    h = (g * jax.lax.logistic(g)) * u
    iota = jax.lax.broadcasted_iota(jnp.int32, (comb_ref.shape[0], N_EXPERTS), 1)
    c_e = jnp.sum(jnp.where(iota == e, comb_ref[...], 0.0),
                  axis=-1, keepdims=True)
    hs = (h * c_e).astype(jnp.bfloat16)
    y = jax.lax.dot_general(hs, wd_ref[0].astype(jnp.bfloat16),
                            (((1,), (1,)), ((), ())),
                            preferred_element_type=jnp.float32)
    out_ref[...] += y


@functools.partial(jax.jit, static_argnums=())
def kernel(x, W_router, W_gate, W_up, W_down):
    Bx, Tx, D = x.shape
    xf = x.reshape(-1, D)
    n = xf.shape[0]
    out, aux = pl.pallas_call(
        _moe_kernel,
        grid=(N_EXPERTS, NF),
        in_specs=[
            pl.BlockSpec((n, D), lambda e, f: (0, 0)),
            pl.BlockSpec((N_EXPERTS, D), lambda e, f: (0, 0)),
            pl.BlockSpec((1, BF, D), lambda e, f: (e, f, 0)),
            pl.BlockSpec((1, BF, D), lambda e, f: (e, f, 0)),
            pl.BlockSpec((1, D, BF), lambda e, f: (e, 0, f)),
        ],
        out_specs=[
            pl.BlockSpec((n, D), lambda e, f: (0, 0)),
            pl.BlockSpec((1, 1), lambda e, f: (0, 0)),
        ],
        out_shape=[
            jax.ShapeDtypeStruct((n, D), jnp.float32),
            jax.ShapeDtypeStruct((1, 1), jnp.float32),
        ],
        scratch_shapes=[pltpu.VMEM((n, N_EXPERTS), jnp.float32)],
        compiler_params=pltpu.CompilerParams(
            dimension_semantics=("arbitrary", "arbitrary"),
        ),
    )(xf, W_router, W_gate, W_up, W_down)
    return out.reshape(Bx, Tx, D), aux[0, 0]
